# baseline (speedup 1.0000x reference)
import functools
import jax
import jax.numpy as jnp
from jax.experimental import pallas as pl
from jax.experimental.pallas import tpu as pltpu

BM = 2048
NBUF = 3
NSPLIT = 4
STRIPE = BM // NSPLIT


def _copy(x_hbm, buf, sems, block, slot):
    for s in range(NSPLIT):
        pltpu.make_async_copy(
            x_hbm.at[pl.ds(block * BM + s * STRIPE, STRIPE), :],
            buf.at[slot, pl.ds(s * STRIPE, STRIPE), :],
            sems.at[slot, s],
        ).start()


def _wait(x_hbm, buf, sems, block, slot):
    for s in range(NSPLIT):
        pltpu.make_async_copy(
            x_hbm.at[pl.ds(block * BM + s * STRIPE, STRIPE), :],
            buf.at[slot, pl.ds(s * STRIPE, STRIPE), :],
            sems.at[slot, s],
        ).wait()


def _tc_body(x_hbm, w_ref, b_ref, o_ref, buf, sems):
    i = pl.program_id(0)
    n = pl.num_programs(0)

    @pl.when(i == 0)
    def _prologue():
        for j in range(NBUF - 1):
            _copy(x_hbm, buf, sems, j, j)

    nxt = i + NBUF - 1

    @pl.when(nxt < n)
    def _prefetch():
        _copy(x_hbm, buf, sems, nxt, nxt % NBUF)

    slot = i % NBUF
    _wait(x_hbm, buf, sems, i, slot)
    wx = jax.lax.dot_general(
        buf[slot], w_ref[...], (((1,), (1,)), ((), ())),
        preferred_element_type=jnp.float32)
    o_ref[...] = jax.nn.sigmoid(wx - b_ref[0])


def kernel(x, input_weights, bias):
    B, K = x.shape
    out = pl.pallas_call(
        _tc_body,
        grid=(B // BM,),
        in_specs=[
            pl.BlockSpec(memory_space=pl.ANY),
            pl.BlockSpec((1, K), lambda i: (0, 0)),
            pl.BlockSpec(memory_space=pltpu.SMEM),
        ],
        out_specs=pl.BlockSpec((BM, 1), lambda i: (i, 0)),
        out_shape=jax.ShapeDtypeStruct((B, 1), jnp.float32),
        scratch_shapes=[
            pltpu.VMEM((NBUF, BM, K), jnp.float32),
            pltpu.SemaphoreType.DMA((NBUF, NSPLIT)),
        ],
    )(x, input_weights, bias)
    return out


# speedup vs baseline: 1.0014x; 1.0014x over previous
import jax
import jax.numpy as jnp
from jax.experimental import pallas as pl
from jax.experimental.pallas import tpu as pltpu

BM = 1024
NSTREAM = 4


def _tc_body(*refs):
    x_refs = refs[:NSTREAM]
    w_ref, b_ref, o_ref = refs[NSTREAM], refs[NSTREAM + 1], refs[NSTREAM + 2]
    w = w_ref[...]
    b = b_ref[0]
    for s in range(NSTREAM):
        wx = jax.lax.dot_general(
            x_refs[s][...], w, (((1,), (1,)), ((), ())),
            preferred_element_type=jnp.float32)
        o_ref[pl.ds(s * BM, BM), :] = jax.nn.sigmoid(wx - b)


def kernel(x, input_weights, bias):
    B, K = x.shape
    grid = B // (BM * NSTREAM)
    x_specs = [
        pl.BlockSpec((BM, K), lambda i, s=s: (i * NSTREAM + s, 0))
        for s in range(NSTREAM)
    ]
    out = pl.pallas_call(
        _tc_body,
        grid=(grid,),
        in_specs=x_specs + [
            pl.BlockSpec((1, K), lambda i: (0, 0)),
            pl.BlockSpec(memory_space=pltpu.SMEM),
        ],
        out_specs=pl.BlockSpec((BM * NSTREAM, 1), lambda i: (i, 0)),
        out_shape=jax.ShapeDtypeStruct((B, 1), jnp.float32),
    )(*([x] * NSTREAM), input_weights, bias)
    return out
